# baseline (speedup 1.0000x reference)
import jax
import jax.numpy as jnp
from jax import lax
from jax.experimental import pallas as pl
from jax.experimental.pallas import tpu as pltpu


def _mask_mul_body(sites_ref, x_ref, o_ref, mask_ref):
    @pl.when(pl.program_id(0) == 0)
    def _():
        s_pad = sites_ref.shape[0]
        w = x_ref.shape[-1]
        sites = sites_ref[...]
        cols = lax.broadcasted_iota(jnp.int32, (s_pad, w), 1)
        hit = jnp.any(sites == cols, axis=0, keepdims=True)
        mask_ref[...] = jnp.where(hit, 0.0, 1.0)

    o_ref[...] = x_ref[...] * mask_ref[...][None]


def kernel(x, mask_sites):
    c, h, w = x.shape
    n = mask_sites.shape[0]
    s_pad = (n + 7) // 8 * 8
    sites = jnp.full((s_pad, 1), w, dtype=jnp.int32)
    sites = lax.dynamic_update_slice(
        sites, mask_sites.astype(jnp.int32).reshape(n, 1), (0, 0)
    )

    bc = 4
    grid = (c // bc,)
    return pl.pallas_call(
        _mask_mul_body,
        grid=grid,
        in_specs=[
            pl.BlockSpec((s_pad, 1), lambda i: (0, 0)),
            pl.BlockSpec((bc, h, w), lambda i: (i, 0, 0)),
        ],
        out_specs=pl.BlockSpec((bc, h, w), lambda i: (i, 0, 0)),
        out_shape=jax.ShapeDtypeStruct((c, h, w), x.dtype),
        scratch_shapes=[pltpu.VMEM((1, w), jnp.float32)],
    )(sites, x)


# speedup vs baseline: 1.0150x; 1.0150x over previous
import jax
import jax.numpy as jnp
from jax import lax
from jax.experimental import pallas as pl
from jax.experimental.pallas import tpu as pltpu


def _mask_mul_body(sites_ref, x_ref, o_ref, mask_ref):
    @pl.when(pl.program_id(0) == 0)
    def _():
        s_pad = sites_ref.shape[0]
        w = x_ref.shape[-1]
        sites = sites_ref[...]
        cols = lax.broadcasted_iota(jnp.int32, (s_pad, w), 1)
        hit = jnp.any(sites == cols, axis=0, keepdims=True)
        mask_ref[...] = jnp.where(hit, 0.0, 1.0)

    o_ref[...] = x_ref[...] * mask_ref[...][None]


def kernel(x, mask_sites):
    c, h, w = x.shape
    n = mask_sites.shape[0]
    s_pad = (n + 7) // 8 * 8
    sites = jnp.full((s_pad, 1), w, dtype=jnp.int32)
    sites = lax.dynamic_update_slice(
        sites, mask_sites.astype(jnp.int32).reshape(n, 1), (0, 0)
    )

    bc = 8
    grid = (c // bc,)
    return pl.pallas_call(
        _mask_mul_body,
        grid=grid,
        in_specs=[
            pl.BlockSpec((s_pad, 1), lambda i: (0, 0)),
            pl.BlockSpec((bc, h, w), lambda i: (i, 0, 0)),
        ],
        out_specs=pl.BlockSpec((bc, h, w), lambda i: (i, 0, 0)),
        out_shape=jax.ShapeDtypeStruct((c, h, w), x.dtype),
        scratch_shapes=[pltpu.VMEM((1, w), jnp.float32)],
    )(sites, x)


# speedup vs baseline: 1.0185x; 1.0035x over previous
import jax
import jax.numpy as jnp
from jax import lax
from jax.experimental import pallas as pl
from jax.experimental.pallas import tpu as pltpu


def _mask_mul_body(sites_ref, x_ref, o_ref, mask_ref):
    @pl.when(pl.program_id(0) == 0)
    def _():
        s_pad = sites_ref.shape[0]
        w = x_ref.shape[-1]
        sites = sites_ref[...]
        cols = lax.broadcasted_iota(jnp.int32, (s_pad, w), 1)
        hit = jnp.any(sites == cols, axis=0, keepdims=True)
        mask_ref[...] = jnp.where(hit, 0.0, 1.0)

    o_ref[...] = x_ref[...] * mask_ref[...][None]


def kernel(x, mask_sites):
    c, h, w = x.shape
    n = mask_sites.shape[0]
    s_pad = (n + 7) // 8 * 8
    sites = jnp.full((s_pad, 1), w, dtype=jnp.int32)
    sites = lax.dynamic_update_slice(
        sites, mask_sites.astype(jnp.int32).reshape(n, 1), (0, 0)
    )

    bc = 12
    grid = (c // bc,)
    return pl.pallas_call(
        _mask_mul_body,
        grid=grid,
        in_specs=[
            pl.BlockSpec((s_pad, 1), lambda i: (0, 0)),
            pl.BlockSpec((bc, h, w), lambda i: (i, 0, 0)),
        ],
        out_specs=pl.BlockSpec((bc, h, w), lambda i: (i, 0, 0)),
        out_shape=jax.ShapeDtypeStruct((c, h, w), x.dtype),
        scratch_shapes=[pltpu.VMEM((1, w), jnp.float32)],
    )(sites, x)
